# baseline (speedup 1.0000x reference)
import functools

import jax
import jax.numpy as jnp
from jax import lax
from jax.experimental import pallas as pl
from jax.experimental.pallas import tpu as pltpu
from jax.experimental.pallas import tpu_sc as plsc

_NC = 2
_NS = 16
_NW = _NC * _NS


def _mlp_body(tab_ref, w1_ref, b1_ref, w2_ref, b2_ref, out_ref):
    h = jnp.dot(tab_ref[...], w1_ref[...], preferred_element_type=jnp.float32)
    h = jnp.maximum(h + b1_ref[...], 0.0)
    o = jnp.dot(h, w2_ref[...], preferred_element_type=jnp.float32)
    out_ref[...] = o + b2_ref[...]


def _mlp_table(table, W1, b1, W2, b2):
    V, D = table.shape
    return pl.pallas_call(
        _mlp_body,
        out_shape=jax.ShapeDtypeStruct((V, D), jnp.float32),
    )(table, W1, b1.reshape(1, D), W2, b2.reshape(1, D))


@functools.lru_cache(maxsize=None)
def _make_gather(V, D, B, chunk):
    assert B % (_NW * chunk) == 0 and chunk % 8 == 0
    b_per_w = B // _NW
    n_chunks = b_per_w // chunk
    mesh = plsc.VectorSubcoreMesh(
        core_axis_name="c", subcore_axis_name="s",
        num_cores=_NC, num_subcores=_NS,
    )

    @functools.partial(
        pl.kernel,
        mesh=mesh,
        out_type=jax.ShapeDtypeStruct((B, D), jnp.float32),
        compiler_params=pltpu.CompilerParams(use_tc_tiling_on_sc=False),
        scratch_types=[
            pltpu.VMEM((b_per_w,), jnp.int32),
            pltpu.VMEM((chunk, D), jnp.float32),
            pltpu.VMEM((chunk, D), jnp.float32),
            pltpu.SemaphoreType.DMA,
            pltpu.SemaphoreType.DMA,
            pltpu.SemaphoreType.DMA,
            pltpu.SemaphoreType.DMA,
        ],
    )
    def gather(tab_hbm, idx_hbm, out_hbm, idx_v, rows0, rows1,
               sg0, sg1, so0, so1):
        wid = lax.axis_index("s") * _NC + lax.axis_index("c")
        base = wid * b_per_w
        rows = (rows0, rows1)
        sg = (sg0, sg1)
        so = (so0, so1)

        pltpu.sync_copy(idx_hbm.at[pl.ds(base, b_per_w)], idx_v)

        def start_gather(g, b):
            return pltpu.async_copy(
                tab_hbm.at[idx_v.at[pl.ds(g * chunk, chunk)]], rows[b], sg[b])

        def start_out(g, b):
            return pltpu.async_copy(
                rows[b], out_hbm.at[pl.ds(base + g * chunk, chunk)], so[b])

        pend_g = {0: start_gather(0, 0)}
        pend_o = {}
        for g in range(n_chunks):
            b = g & 1
            if g + 1 < n_chunks:
                if g >= 1:
                    pend_o.pop(g - 1).wait()
                pend_g[g + 1] = start_gather(g + 1, 1 - b)
            pend_g.pop(g).wait()
            pend_o[g] = start_out(g, b)
        for g in sorted(pend_o):
            pend_o.pop(g).wait()

    return gather


def kernel(emotion_ids, table, W1, b1, W2, b2):
    Bb, H = emotion_ids.shape
    V, D = table.shape
    mlp_tab = _mlp_table(table, W1, b1, W2, b2)
    flat_idx = emotion_ids.reshape(-1).astype(jnp.int32)
    out_flat = _make_gather(V, D, Bb * H, 512)(mlp_tab, flat_idx)
    return out_flat.reshape(Bb, H, D)
